# baseline (speedup 1.0000x reference)
import functools

import jax
import jax.numpy as jnp
from jax import lax
from jax.experimental import pallas as pl
from jax.experimental.pallas import tpu as pltpu
from jax.experimental.pallas import tpu_sc as plsc

M = 4096
K = 128
N = 100000
NB = 512
N_PAD = ((N + NB - 1) // NB) * NB
NUM_NBLK = N_PAD // NB

SC_NC = 2
SC_NS = 16
SC_NW = SC_NC * SC_NS
B_PER_W = M // SC_NW
DPAD = 128


def _argmin_body(q_ref, qsq_ref, v_ref, vsq_ref, idx_out, best_val, best_idx):
    j = pl.program_id(0)

    @pl.when(j == 0)
    def _init():
        best_val[...] = jnp.full_like(best_val[...], jnp.inf)
        best_idx[...] = jnp.zeros_like(best_idx[...])

    q = q_ref[...]
    v = v_ref[...]
    qv = lax.dot_general(q, v, (((1,), (1,)), ((), ())),
                         preferred_element_type=jnp.float32)
    dist = (qsq_ref[...] + vsq_ref[...]) - 2.0 * qv
    local_min = jnp.min(dist, axis=1, keepdims=True)
    col = j * NB + lax.broadcasted_iota(jnp.int32, (M, NB), 1)
    local_arg = jnp.min(
        jnp.where(dist == local_min, col, jnp.int32(2**30)),
        axis=1, keepdims=True)
    better = local_min < best_val[...]
    best_val[...] = jnp.where(better, local_min, best_val[...])
    best_idx[...] = jnp.where(better, local_arg, best_idx[...])

    @pl.when(j == NUM_NBLK - 1)
    def _out():
        idx_out[...] = best_idx[...]


_argmin_call = pl.pallas_call(
    _argmin_body,
    grid=(NUM_NBLK,),
    in_specs=[
        pl.BlockSpec((M, K), lambda j: (0, 0)),
        pl.BlockSpec((M, 1), lambda j: (0, 0)),
        pl.BlockSpec((NB, K), lambda j: (j, 0)),
        pl.BlockSpec((1, NB), lambda j: (0, j)),
    ],
    out_specs=pl.BlockSpec((M, 1), lambda j: (0, 0)),
    out_shape=jax.ShapeDtypeStruct((M, 1), jnp.int32),
    scratch_shapes=[
        pltpu.VMEM((M, 1), jnp.float32),
        pltpu.VMEM((M, 1), jnp.int32),
    ],
)


@functools.cache
def _make_gather_colors():
    @functools.partial(
        pl.kernel,
        mesh=plsc.VectorSubcoreMesh(core_axis_name="c", subcore_axis_name="s"),
        out_type=jax.ShapeDtypeStruct((M, DPAD), jnp.float32),
        scratch_types=[
            pltpu.VMEM((B_PER_W,), jnp.int32),
            pltpu.VMEM((B_PER_W, DPAD), jnp.float32),
            pltpu.SemaphoreType.DMA,
        ],
    )
    def _gather_colors(table_hbm, idx_hbm, out_hbm, idx_v, rows_v, sem):
        wid = lax.axis_index("s") * SC_NC + lax.axis_index("c")
        base = wid * B_PER_W
        pltpu.sync_copy(idx_hbm.at[pl.ds(base, B_PER_W)], idx_v)
        pltpu.async_copy(table_hbm.at[idx_v], rows_v, sem).wait()
        pltpu.sync_copy(rows_v, out_hbm.at[pl.ds(base, B_PER_W)])

    return _gather_colors


def kernel(cse_embedding, verts_cse_embedding, verts_colors):
    q_sq = jnp.sum(cse_embedding * cse_embedding, axis=1, keepdims=True)
    v_sq = jnp.sum(verts_cse_embedding * verts_cse_embedding, axis=1)
    v_sq = jnp.pad(v_sq, (0, N_PAD - N), constant_values=jnp.inf)[None, :]
    def _rn8(x):
        t = x * jnp.float32(65537.0)
        return t - (t - x)

    q_bf = _rn8(cse_embedding).astype(jnp.bfloat16)
    v_pad = _rn8(jnp.pad(verts_cse_embedding, ((0, N_PAD - N), (0, 0)))).astype(
        jnp.bfloat16)

    closest = _argmin_call(q_bf, q_sq, v_pad, v_sq)[:, 0]

    colors_pad = jnp.pad(verts_colors, ((0, 0), (0, DPAD - 3)))
    out = _make_gather_colors()(colors_pad, closest)
    return out[:, :3]


# speedup vs baseline: 1.2111x; 1.2111x over previous
import functools

import jax
import jax.numpy as jnp
from jax import lax
from jax.experimental import pallas as pl
from jax.experimental.pallas import tpu as pltpu
from jax.experimental.pallas import tpu_sc as plsc

M = 4096
K = 128
N = 100000
NB = 512
N_PAD = ((N + NB - 1) // NB) * NB
NUM_NBLK = N_PAD // NB

SC_NC = 2
SC_NS = 16
SC_NW = SC_NC * SC_NS
B_PER_W = M // SC_NW
DPAD = 128


def _argmin_body(q_ref, v_ref, hvsq_ref, idx_out, best_val, best_idx):
    j = pl.program_id(0)

    @pl.when(j == 0)
    def _init():
        best_val[...] = jnp.full_like(best_val[...], -jnp.inf)
        best_idx[...] = jnp.zeros_like(best_idx[...])

    q = q_ref[...]
    v = v_ref[...]
    qv = lax.dot_general(q, v, (((1,), (1,)), ((), ())),
                         preferred_element_type=jnp.float32)
    score = qv - hvsq_ref[...]
    local_max = jnp.max(score, axis=1, keepdims=True)
    col = j * NB + lax.broadcasted_iota(jnp.int32, (M, NB), 1)
    local_arg = jnp.min(
        jnp.where(score == local_max, col, jnp.int32(2**30)),
        axis=1, keepdims=True)
    better = local_max > best_val[...]
    best_val[...] = jnp.where(better, local_max, best_val[...])
    best_idx[...] = jnp.where(better, local_arg, best_idx[...])

    @pl.when(j == NUM_NBLK - 1)
    def _out():
        idx_out[...] = best_idx[...]


_argmin_call = pl.pallas_call(
    _argmin_body,
    grid=(NUM_NBLK,),
    in_specs=[
        pl.BlockSpec((M, K), lambda j: (0, 0)),
        pl.BlockSpec((NB, K), lambda j: (j, 0)),
        pl.BlockSpec((1, NB), lambda j: (0, j)),
    ],
    out_specs=pl.BlockSpec((M, 1), lambda j: (0, 0)),
    out_shape=jax.ShapeDtypeStruct((M, 1), jnp.int32),
    scratch_shapes=[
        pltpu.VMEM((M, 1), jnp.float32),
        pltpu.VMEM((M, 1), jnp.int32),
    ],
)


@functools.cache
def _make_gather_colors():
    @functools.partial(
        pl.kernel,
        mesh=plsc.VectorSubcoreMesh(core_axis_name="c", subcore_axis_name="s"),
        out_type=jax.ShapeDtypeStruct((M, DPAD), jnp.float32),
        scratch_types=[
            pltpu.VMEM((B_PER_W,), jnp.int32),
            pltpu.VMEM((B_PER_W, DPAD), jnp.float32),
            pltpu.SemaphoreType.DMA,
        ],
    )
    def _gather_colors(table_hbm, idx_hbm, out_hbm, idx_v, rows_v, sem):
        wid = lax.axis_index("s") * SC_NC + lax.axis_index("c")
        base = wid * B_PER_W
        pltpu.sync_copy(idx_hbm.at[pl.ds(base, B_PER_W)], idx_v)
        pltpu.async_copy(table_hbm.at[idx_v], rows_v, sem).wait()
        pltpu.sync_copy(rows_v, out_hbm.at[pl.ds(base, B_PER_W)])

    return _gather_colors


def kernel(cse_embedding, verts_cse_embedding, verts_colors):
    v_sq = jnp.sum(verts_cse_embedding * verts_cse_embedding, axis=1)
    half_vsq = jnp.pad(0.5 * v_sq, (0, N_PAD - N),
                       constant_values=jnp.inf)[None, :]
    q_bf = cse_embedding.astype(jnp.bfloat16)
    v_pad = jnp.pad(verts_cse_embedding, ((0, N_PAD - N), (0, 0))).astype(
        jnp.bfloat16)

    closest = _argmin_call(q_bf, v_pad, half_vsq)[:, 0]

    colors_pad = jnp.pad(verts_colors, ((0, 0), (0, DPAD - 3)))
    out = _make_gather_colors()(colors_pad, closest)
    return out[:, :3]


# speedup vs baseline: 1.2789x; 1.0560x over previous
import functools

import jax
import jax.numpy as jnp
from jax import lax
from jax.experimental import pallas as pl
from jax.experimental.pallas import tpu as pltpu
from jax.experimental.pallas import tpu_sc as plsc

M = 4096
K = 128
N = 100000
NB = 1024
N_PAD = ((N + NB - 1) // NB) * NB
NUM_NBLK = N_PAD // NB

SC_NC = 2
SC_NS = 16
SC_NW = SC_NC * SC_NS
B_PER_W = M // SC_NW
DPAD = 128


def _argmin_body(q_ref, v_ref, hvsq_ref, idx_out, best_val, best_idx):
    j = pl.program_id(0)

    @pl.when(j == 0)
    def _init():
        best_val[...] = jnp.full_like(best_val[...], -jnp.inf)
        best_idx[...] = jnp.zeros_like(best_idx[...])

    q = q_ref[...]
    v = v_ref[...]
    qv = lax.dot_general(q, v, (((1,), (1,)), ((), ())),
                         preferred_element_type=jnp.float32)
    score = qv - hvsq_ref[...]
    local_max = jnp.max(score, axis=1, keepdims=True)
    col = j * NB + lax.broadcasted_iota(jnp.int32, (M, NB), 1)
    local_arg = jnp.min(
        jnp.where(score == local_max, col, jnp.int32(2**30)),
        axis=1, keepdims=True)
    better = local_max > best_val[...]
    best_val[...] = jnp.where(better, local_max, best_val[...])
    best_idx[...] = jnp.where(better, local_arg, best_idx[...])

    @pl.when(j == NUM_NBLK - 1)
    def _out():
        idx_out[...] = best_idx[...]


_argmin_call = pl.pallas_call(
    _argmin_body,
    grid=(NUM_NBLK,),
    in_specs=[
        pl.BlockSpec((M, K), lambda j: (0, 0)),
        pl.BlockSpec((NB, K), lambda j: (j, 0)),
        pl.BlockSpec((1, NB), lambda j: (0, j)),
    ],
    out_specs=pl.BlockSpec((M, 1), lambda j: (0, 0)),
    out_shape=jax.ShapeDtypeStruct((M, 1), jnp.int32),
    scratch_shapes=[
        pltpu.VMEM((M, 1), jnp.float32),
        pltpu.VMEM((M, 1), jnp.int32),
    ],
)


@functools.cache
def _make_gather_colors():
    @functools.partial(
        pl.kernel,
        mesh=plsc.VectorSubcoreMesh(core_axis_name="c", subcore_axis_name="s"),
        out_type=jax.ShapeDtypeStruct((M, DPAD), jnp.float32),
        scratch_types=[
            pltpu.VMEM((B_PER_W,), jnp.int32),
            pltpu.VMEM((B_PER_W, DPAD), jnp.float32),
            pltpu.SemaphoreType.DMA,
        ],
    )
    def _gather_colors(table_hbm, idx_hbm, out_hbm, idx_v, rows_v, sem):
        wid = lax.axis_index("s") * SC_NC + lax.axis_index("c")
        base = wid * B_PER_W
        pltpu.sync_copy(idx_hbm.at[pl.ds(base, B_PER_W)], idx_v)
        pltpu.async_copy(table_hbm.at[idx_v], rows_v, sem).wait()
        pltpu.sync_copy(rows_v, out_hbm.at[pl.ds(base, B_PER_W)])

    return _gather_colors


def kernel(cse_embedding, verts_cse_embedding, verts_colors):
    v_sq = jnp.sum(verts_cse_embedding * verts_cse_embedding, axis=1)
    half_vsq = jnp.pad(0.5 * v_sq, (0, N_PAD - N),
                       constant_values=jnp.inf)[None, :]
    q_bf = cse_embedding.astype(jnp.bfloat16)
    v_pad = jnp.pad(verts_cse_embedding, ((0, N_PAD - N), (0, 0))).astype(
        jnp.bfloat16)

    closest = _argmin_call(q_bf, v_pad, half_vsq)[:, 0]

    colors_pad = jnp.pad(verts_colors, ((0, 0), (0, DPAD - 3)))
    out = _make_gather_colors()(colors_pad, closest)
    return out[:, :3]


# speedup vs baseline: 1.3084x; 1.0230x over previous
import functools

import jax
import jax.numpy as jnp
from jax import lax
from jax.experimental import pallas as pl
from jax.experimental.pallas import tpu as pltpu
from jax.experimental.pallas import tpu_sc as plsc

M = 4096
K = 128
N = 100000
NB = 2048
N_PAD = ((N + NB - 1) // NB) * NB
NUM_NBLK = N_PAD // NB

SC_NC = 2
SC_NS = 16
SC_NW = SC_NC * SC_NS
B_PER_W = M // SC_NW
DPAD = 128


def _argmin_body(q_ref, v_ref, hvsq_ref, idx_out, best_val, best_idx):
    j = pl.program_id(0)

    @pl.when(j == 0)
    def _init():
        best_val[...] = jnp.full_like(best_val[...], -jnp.inf)
        best_idx[...] = jnp.zeros_like(best_idx[...])

    q = q_ref[...]
    v = v_ref[...]
    qv = lax.dot_general(q, v, (((1,), (1,)), ((), ())),
                         preferred_element_type=jnp.float32)
    score = qv - hvsq_ref[...]
    local_max = jnp.max(score, axis=1, keepdims=True)
    col = j * NB + lax.broadcasted_iota(jnp.int32, (M, NB), 1)
    local_arg = jnp.min(
        jnp.where(score == local_max, col, jnp.int32(2**30)),
        axis=1, keepdims=True)
    better = local_max > best_val[...]
    best_val[...] = jnp.where(better, local_max, best_val[...])
    best_idx[...] = jnp.where(better, local_arg, best_idx[...])

    @pl.when(j == NUM_NBLK - 1)
    def _out():
        idx_out[...] = best_idx[...]


_argmin_call = pl.pallas_call(
    _argmin_body,
    grid=(NUM_NBLK,),
    in_specs=[
        pl.BlockSpec((M, K), lambda j: (0, 0)),
        pl.BlockSpec((NB, K), lambda j: (j, 0)),
        pl.BlockSpec((1, NB), lambda j: (0, j)),
    ],
    out_specs=pl.BlockSpec((M, 1), lambda j: (0, 0)),
    out_shape=jax.ShapeDtypeStruct((M, 1), jnp.int32),
    scratch_shapes=[
        pltpu.VMEM((M, 1), jnp.float32),
        pltpu.VMEM((M, 1), jnp.int32),
    ],
)


@functools.cache
def _make_gather_colors():
    @functools.partial(
        pl.kernel,
        mesh=plsc.VectorSubcoreMesh(core_axis_name="c", subcore_axis_name="s"),
        out_type=jax.ShapeDtypeStruct((M, DPAD), jnp.float32),
        scratch_types=[
            pltpu.VMEM((B_PER_W,), jnp.int32),
            pltpu.VMEM((B_PER_W, DPAD), jnp.float32),
            pltpu.SemaphoreType.DMA,
        ],
    )
    def _gather_colors(table_hbm, idx_hbm, out_hbm, idx_v, rows_v, sem):
        wid = lax.axis_index("s") * SC_NC + lax.axis_index("c")
        base = wid * B_PER_W
        pltpu.sync_copy(idx_hbm.at[pl.ds(base, B_PER_W)], idx_v)
        pltpu.async_copy(table_hbm.at[idx_v], rows_v, sem).wait()
        pltpu.sync_copy(rows_v, out_hbm.at[pl.ds(base, B_PER_W)])

    return _gather_colors


def kernel(cse_embedding, verts_cse_embedding, verts_colors):
    v_sq = jnp.sum(verts_cse_embedding * verts_cse_embedding, axis=1)
    half_vsq = jnp.pad(0.5 * v_sq, (0, N_PAD - N),
                       constant_values=jnp.inf)[None, :]
    q_bf = cse_embedding.astype(jnp.bfloat16)
    v_pad = jnp.pad(verts_cse_embedding, ((0, N_PAD - N), (0, 0))).astype(
        jnp.bfloat16)

    closest = _argmin_call(q_bf, v_pad, half_vsq)[:, 0]

    colors_pad = jnp.pad(verts_colors, ((0, 0), (0, DPAD - 3)))
    out = _make_gather_colors()(colors_pad, closest)
    return out[:, :3]
